# baseline (speedup 1.0000x reference)
import functools

import jax
import jax.numpy as jnp
from jax import lax
from jax.experimental import pallas as pl
from jax.experimental.pallas import tpu as pltpu
from jax.experimental.pallas import tpu_sc as plsc

NC = 2
NS = 16
LANES = 16
CH = 80


@functools.cache
def _make_scatter(n_nodes, d, n_edges):
    half = n_nodes // NC
    acc_rows = 5120
    e_per_t = n_edges // NS
    assert e_per_t * NS == n_edges and e_per_t % CH == 0
    n_ch = e_per_t // CH
    SC_CH = 25
    n_sch = n_ch // SC_CH
    assert n_sch * SC_CH == n_ch
    n_dj = d // LANES
    n_g = CH // LANES

    mesh = plsc.VectorSubcoreMesh(core_axis_name="c", subcore_axis_name="s")

    @functools.partial(
        pl.kernel,
        out_type=jax.ShapeDtypeStruct((n_nodes, d), jnp.float32),
        mesh=mesh,
        scratch_types=[
            pltpu.VMEM((SC_CH * CH,), jnp.int32),
            pltpu.VMEM((SC_CH * CH,), jnp.int32),
            pltpu.VMEM((SC_CH * CH,), jnp.float32),
            pltpu.VMEM((4, CH), jnp.int32),
            pltpu.VMEM((4, CH), jnp.int32),
            pltpu.VMEM((4 * CH, d), jnp.float32),
            pltpu.VMEM_SHARED((acc_rows, d), jnp.float32),
            pltpu.SemaphoreType.DMA,
            pltpu.SemaphoreType.DMA,
            pltpu.SemaphoreType.DMA,
        ],
    )
    def scatter_kernel(emb_hbm, src_hbm, dst_hbm, w_hbm, out_hbm,
                       src_v, dst_v, w_v, midx_v, gidx_v, rows_v, acc_sh,
                       gsem, ssem_a, ssem_b):
        cid = lax.axis_index("c")
        sid = lax.axis_index("s")
        base = cid * half

        def gather_desc(ridx, slot0):
            return pltpu.make_async_copy(
                emb_hbm.at[gidx_v.at[ridx]],
                rows_v.at[pl.ds(slot0, CH)], gsem)

        def prep(kp):
            bp = lax.rem(kp, 4)
            k0p = kp * CH
            for g in range(n_g):
                sl16 = pl.ds(g * LANES, LANES)
                d16 = dst_v[pl.ds(k0p + g * LANES, LANES)]
                s16 = src_v[pl.ds(k0p + g * LANES, LANES)]
                loc = d16 - base
                sel = (loc >= 0) & (loc < half)
                midx_v[bp, sl16] = jnp.where(sel, loc, half)
                gidx_v[bp, sl16] = jnp.where(sel, s16, 0)

        def scatter_desc(slot0, ridx, sem):
            return pltpu.make_async_copy(
                rows_v.at[pl.ds(slot0, CH)], acc_sh.at[midx_v.at[ridx]],
                sem)

        zvec = jnp.zeros((LANES,), jnp.float32)

        def zfill(i, _):
            for j in range(n_dj):
                rows_v[i, pl.ds(j * LANES, LANES)] = zvec
            return 0

        lax.fori_loop(0, CH, zfill, 0)
        for z in range(4):
            pltpu.sync_copy(rows_v.at[pl.ds(0, CH)],
                            acc_sh.at[pl.ds(sid * 320 + z * CH, CH)])

        plsc.subcore_barrier()

        def superchunk(cs, _):
            s0 = sid * e_per_t + cs * (SC_CH * CH)
            pltpu.sync_copy(src_hbm.at[pl.ds(s0, SC_CH * CH)], src_v)
            pltpu.sync_copy(dst_hbm.at[pl.ds(s0, SC_CH * CH)], dst_v)
            pltpu.sync_copy(w_hbm.at[pl.ds(s0, SC_CH * CH)], w_v)

            prep(0)
            prep(1)
            gather_desc(0, 0).start()
            gather_desc(1, CH).start()

            def chunk(k, _):
                b = lax.rem(k, 4)
                bb = b * CH
                par = lax.rem(k, 2)
                gather_desc(0, 0).wait()

                @pl.when((k >= 2) & (par == 0))
                def _wa():
                    scatter_desc(0, 0, ssem_a).wait()

                @pl.when((k >= 2) & (par == 1))
                def _wb():
                    scatter_desc(0, 0, ssem_b).wait()

                @pl.when(k < SC_CH - 2)
                def _pf():
                    prep(k + 2)
                    gather_desc(lax.rem(k + 2, 4),
                                lax.rem(k + 2, 4) * CH).start()

                k0 = k * CH
                for g in range(n_g):
                    w16 = w_v[pl.ds(k0 + g * LANES, LANES)]
                    for i in range(LANES):
                        w = w16[i]
                        e = bb + g * LANES + i
                        for j in range(n_dj):
                            sl = pl.ds(j * LANES, LANES)
                            rows_v[e, sl] = rows_v[e, sl] * w

                @pl.when(par == 0)
                def _sa():
                    pltpu.async_copy(rows_v.at[pl.ds(bb, CH)],
                                     acc_sh.at[midx_v.at[b]], ssem_a,
                                     add=True)

                @pl.when(par == 1)
                def _sb():
                    pltpu.async_copy(rows_v.at[pl.ds(bb, CH)],
                                     acc_sh.at[midx_v.at[b]], ssem_b,
                                     add=True)

                return 0

            lax.fori_loop(0, SC_CH, chunk, 0)
            scatter_desc(0, 0, ssem_a).wait()
            scatter_desc(0, 0, ssem_b).wait()
            return 0

        lax.fori_loop(0, n_sch, superchunk, 0)

        plsc.subcore_barrier()
        @pl.when(sid < 12)
        def _dump():
            d0 = sid * 416
            pltpu.sync_copy(acc_sh.at[pl.ds(d0, 416)],
                            out_hbm.at[pl.ds(base + d0, 416)])

        @pl.when(sid == 12)
        def _dump_tail():
            pltpu.sync_copy(acc_sh.at[pl.ds(4992, 8)],
                            out_hbm.at[pl.ds(base + 4992, 8)])

    return scatter_kernel


def _dense_layer(all_emb, side, W1, b1, W2, b2, acc_e, acc_s, inv_layers):
    n, d = all_emb.shape
    blk = 1000
    grid = n // blk

    def body(emb_ref, s_ref, w1_ref, b1_ref, w2_ref, b2_ref,
             ae_ref, as_ref, ne_ref, aeo_ref, aso_ref):
        side_b = s_ref[...]
        emb = emb_ref[...]
        h1 = jnp.dot(emb + side_b, w1_ref[...],
                     preferred_element_type=jnp.float32) + b1_ref[...]
        h1 = jnp.where(h1 >= 0, h1, 0.01 * h1)
        h2 = jnp.dot(emb * side_b, w2_ref[...],
                     preferred_element_type=jnp.float32) + b2_ref[...]
        h2 = jnp.where(h2 >= 0, h2, 0.01 * h2)
        e = h1 + h2
        nrm = jnp.maximum(
            jnp.sqrt(jnp.sum(e * e, axis=1, keepdims=True)), 1e-12)
        ne = e / nrm
        ne_ref[...] = ne
        aeo_ref[...] = ae_ref[...] + ne * inv_layers
        aso_ref[...] = as_ref[...] + side_b * inv_layers

    bspec = pl.BlockSpec((blk, d), lambda i: (i, 0))
    wspec = pl.BlockSpec((d, d), lambda i: (0, 0))
    b_spec = pl.BlockSpec((1, d), lambda i: (0, 0))
    out_sd = jax.ShapeDtypeStruct((n, d), jnp.float32)
    return pl.pallas_call(
        body,
        grid=(grid,),
        in_specs=[bspec, bspec, wspec, b_spec, wspec, b_spec, bspec, bspec],
        out_specs=[bspec, bspec, bspec],
        out_shape=[out_sd, out_sd, out_sd],
    )(all_emb, side, W1, b1.reshape(1, d), W2, b2.reshape(1, d),
      acc_e, acc_s)


def kernel(items_emb, edge_index, edge_weight, W1, b1, W2, b2):
    n_items, d = items_emb.shape
    n_nodes = 2 * n_items
    n_edges = edge_index.shape[1]
    n_layers = 3

    all_emb = jnp.concatenate([items_emb, items_emb], axis=0)
    src3 = edge_index[0].astype(jnp.int32)
    dst3 = edge_index[1].astype(jnp.int32)
    w3 = edge_weight

    scatter = _make_scatter(n_nodes, d, n_edges)
    acc0 = jnp.zeros((n_nodes, d), jnp.float32)

    def layer(_, carry):
        emb, acc_e, acc_s = carry
        side = scatter(emb, src3, dst3, w3)
        return tuple(_dense_layer(emb, side, W1, b1, W2, b2, acc_e, acc_s,
                                  1.0 / n_layers))

    _, acc_e, acc_s = lax.fori_loop(0, n_layers, layer,
                                    (all_emb, acc0, acc0))
    return (acc_e[:n_items], acc_e[n_items:],
            acc_s[:n_items], acc_s[n_items:], items_emb)


# speedup vs baseline: 35.3231x; 35.3231x over previous
import functools

import jax
import jax.numpy as jnp
from jax import lax
from jax.experimental import pallas as pl
from jax.experimental.pallas import tpu as pltpu
from jax.experimental.pallas import tpu_sc as plsc

NC = 2
NS = 16
LANES = 16
CH = 80


@functools.cache
def _make_scatter(n_nodes, d, n_edges):
    half = n_nodes // NC
    acc_rows = 5120
    e_per_t = n_edges // NS
    assert e_per_t * NS == n_edges and e_per_t % CH == 0
    n_ch = e_per_t // CH
    SC_CH = 25
    n_sch = n_ch // SC_CH
    assert n_sch * SC_CH == n_ch
    n_dj = d // LANES
    n_g = CH // LANES

    mesh = plsc.VectorSubcoreMesh(core_axis_name="c", subcore_axis_name="s")

    @functools.partial(
        pl.kernel,
        out_type=jax.ShapeDtypeStruct((n_nodes, d), jnp.float32),
        mesh=mesh,
        scratch_types=[
            pltpu.VMEM((SC_CH * CH,), jnp.int32),
            pltpu.VMEM((SC_CH * CH,), jnp.int32),
            pltpu.VMEM((SC_CH * CH,), jnp.float32),
            pltpu.VMEM((4, CH), jnp.int32),
            pltpu.VMEM((4 * CH, d), jnp.float32),
            pltpu.VMEM_SHARED((acc_rows, d), jnp.float32),
            pltpu.SemaphoreType.DMA,
            pltpu.SemaphoreType.DMA,
            pltpu.SemaphoreType.DMA,
        ],
    )
    def scatter_kernel(emb_hbm, src_hbm, dst_hbm, w_hbm, out_hbm,
                       src_v, dst_v, w_v, midx_v, rows_v, acc_sh,
                       gsem, ssem_a, ssem_b):
        cid = lax.axis_index("c")
        sid = lax.axis_index("s")
        base = cid * half

        def gather_desc(k0_idx, slot0):
            return pltpu.make_async_copy(
                emb_hbm.at[src_v.at[pl.ds(k0_idx, CH)]],
                rows_v.at[pl.ds(slot0, CH)], gsem)

        def scatter_desc(slot0, ridx, sem):
            return pltpu.make_async_copy(
                rows_v.at[pl.ds(slot0, CH)], acc_sh.at[midx_v.at[ridx]],
                sem)

        zvec = jnp.zeros((LANES,), jnp.float32)

        def zfill(i, _):
            for j in range(n_dj):
                rows_v[i, pl.ds(j * LANES, LANES)] = zvec
            return 0

        lax.fori_loop(0, CH, zfill, 0)
        for z in range(4):
            pltpu.sync_copy(rows_v.at[pl.ds(0, CH)],
                            acc_sh.at[pl.ds(sid * 320 + z * CH, CH)])

        plsc.subcore_barrier()

        def superchunk(cs, _):
            s0 = sid * e_per_t + cs * (SC_CH * CH)
            pltpu.sync_copy(src_hbm.at[pl.ds(s0, SC_CH * CH)], src_v)
            pltpu.sync_copy(dst_hbm.at[pl.ds(s0, SC_CH * CH)], dst_v)
            pltpu.sync_copy(w_hbm.at[pl.ds(s0, SC_CH * CH)], w_v)

            gather_desc(0, 0).start()
            gather_desc(CH, CH).start()

            def chunk(k, _):
                b = lax.rem(k, 4)
                bb = b * CH
                par = lax.rem(k, 2)
                gather_desc(0, 0).wait()

                @pl.when((k >= 2) & (par == 0))
                def _wa():
                    scatter_desc(0, 0, ssem_a).wait()

                @pl.when((k >= 2) & (par == 1))
                def _wb():
                    scatter_desc(0, 0, ssem_b).wait()

                @pl.when(k < SC_CH - 2)
                def _pf():
                    gather_desc((k + 2) * CH,
                                lax.rem(k + 2, 4) * CH).start()

                k0 = k * CH
                for g in range(n_g):
                    sl16 = pl.ds(g * LANES, LANES)
                    d16 = dst_v[pl.ds(k0 + g * LANES, LANES)]
                    loc = d16 - base
                    sel = (loc >= 0) & (loc < half)
                    midx_v[b, sl16] = jnp.where(sel, loc, half)
                    w16 = w_v[pl.ds(k0 + g * LANES, LANES)]
                    for i in range(LANES):
                        w = w16[i]
                        e = bb + g * LANES + i
                        for j in range(n_dj):
                            sl = pl.ds(j * LANES, LANES)
                            rows_v[e, sl] = rows_v[e, sl] * w

                @pl.when(par == 0)
                def _sa():
                    pltpu.async_copy(rows_v.at[pl.ds(bb, CH)],
                                     acc_sh.at[midx_v.at[b]], ssem_a,
                                     add=True)

                @pl.when(par == 1)
                def _sb():
                    pltpu.async_copy(rows_v.at[pl.ds(bb, CH)],
                                     acc_sh.at[midx_v.at[b]], ssem_b,
                                     add=True)

                return 0

            lax.fori_loop(0, SC_CH, chunk, 0)
            scatter_desc(0, 0, ssem_a).wait()
            scatter_desc(0, 0, ssem_b).wait()
            return 0

        lax.fori_loop(0, n_sch, superchunk, 0)

        plsc.subcore_barrier()
        @pl.when(sid < 12)
        def _dump():
            d0 = sid * 416
            pltpu.sync_copy(acc_sh.at[pl.ds(d0, 416)],
                            out_hbm.at[pl.ds(base + d0, 416)])

        @pl.when(sid == 12)
        def _dump_tail():
            pltpu.sync_copy(acc_sh.at[pl.ds(4992, 8)],
                            out_hbm.at[pl.ds(base + 4992, 8)])

    return scatter_kernel


def _dense_layer(all_emb, side, W1, b1, W2, b2, acc_e, acc_s, inv_layers):
    n, d = all_emb.shape
    blk = 1000
    grid = n // blk

    def body(emb_ref, s_ref, w1_ref, b1_ref, w2_ref, b2_ref,
             ae_ref, as_ref, ne_ref, aeo_ref, aso_ref):
        side_b = s_ref[...]
        emb = emb_ref[...]
        h1 = jnp.dot(emb + side_b, w1_ref[...],
                     preferred_element_type=jnp.float32) + b1_ref[...]
        h1 = jnp.where(h1 >= 0, h1, 0.01 * h1)
        h2 = jnp.dot(emb * side_b, w2_ref[...],
                     preferred_element_type=jnp.float32) + b2_ref[...]
        h2 = jnp.where(h2 >= 0, h2, 0.01 * h2)
        e = h1 + h2
        nrm = jnp.maximum(
            jnp.sqrt(jnp.sum(e * e, axis=1, keepdims=True)), 1e-12)
        ne = e / nrm
        ne_ref[...] = ne
        aeo_ref[...] = ae_ref[...] + ne * inv_layers
        aso_ref[...] = as_ref[...] + side_b * inv_layers

    bspec = pl.BlockSpec((blk, d), lambda i: (i, 0))
    wspec = pl.BlockSpec((d, d), lambda i: (0, 0))
    b_spec = pl.BlockSpec((1, d), lambda i: (0, 0))
    out_sd = jax.ShapeDtypeStruct((n, d), jnp.float32)
    return pl.pallas_call(
        body,
        grid=(grid,),
        in_specs=[bspec, bspec, wspec, b_spec, wspec, b_spec, bspec, bspec],
        out_specs=[bspec, bspec, bspec],
        out_shape=[out_sd, out_sd, out_sd],
    )(all_emb, side, W1, b1.reshape(1, d), W2, b2.reshape(1, d),
      acc_e, acc_s)


def kernel(items_emb, edge_index, edge_weight, W1, b1, W2, b2):
    n_items, d = items_emb.shape
    n_nodes = 2 * n_items
    n_edges = edge_index.shape[1]
    n_layers = 3

    all_emb = jnp.concatenate([items_emb, items_emb], axis=0)
    src3 = edge_index[0].astype(jnp.int32)
    dst3 = edge_index[1].astype(jnp.int32)
    w3 = edge_weight

    scatter = _make_scatter(n_nodes, d, n_edges)
    acc0 = jnp.zeros((n_nodes, d), jnp.float32)

    def layer(_, carry):
        emb, acc_e, acc_s = carry
        side = scatter(emb, src3, dst3, w3)
        return tuple(_dense_layer(emb, side, W1, b1, W2, b2, acc_e, acc_s,
                                  1.0 / n_layers))

    _, acc_e, acc_s = lax.fori_loop(0, n_layers, layer,
                                    (all_emb, acc0, acc0))
    return (acc_e[:n_items], acc_e[n_items:],
            acc_s[:n_items], acc_s[n_items:], items_emb)
